# baseline (speedup 1.0000x reference)
import jax
import jax.numpy as jnp
from jax.experimental import pallas as pl
from jax.experimental.pallas import tpu as pltpu

_NC = 4
_M = 4096
_D = 64
_B = 4608
_TB = 512
_NB = _B // _TB
_COMMITMENT_COST = 0.25


def _vq_block(x_ref, e_ref, xsq_ref, esq_ref,
              q_ref, idx_ref, counts_ref, commit_ref, cb_ref, perp_ref):
    i = pl.program_id(0)
    xb = x_ref[...]
    loss_part = jnp.zeros((1, 1), dtype=jnp.float32)
    idx_rows = []
    counts_rows = []
    for c in range(_NC):
        xc = xb[:, c, :]
        ec = e_ref[c]
        cross = jax.lax.dot_general(
            xc, ec, (((1,), (1,)), ((), ())),
            preferred_element_type=jnp.float32)
        dist = (xsq_ref[:, c:c + 1] - 2.0 * cross) + esq_ref[c][None, :]
        minv = jnp.min(dist, axis=-1)
        iota = jax.lax.broadcasted_iota(jnp.int32, (_TB, _M), 1)
        idx_c = jnp.min(jnp.where(dist == minv[:, None], iota, _M), axis=-1)
        loss_part += jnp.sum(minv)[None, None]
        onehot = (iota == idx_c[:, None]).astype(jnp.float32)
        qc = jax.lax.dot_general(
            onehot, ec, (((1,), (0,)), ((), ())),
            preferred_element_type=jnp.float32,
            precision=jax.lax.Precision.HIGHEST)
        q_ref[:, c, :] = qc
        idx_rows.append(idx_c)
        counts_rows.append(jnp.sum(onehot, axis=0))
    idx_ref[...] = jnp.stack(idx_rows, axis=0)
    counts_new = jnp.stack(counts_rows, axis=0)

    @pl.when(i == 0)
    def _():
        counts_ref[...] = counts_new
        commit_ref[...] = loss_part

    @pl.when(i > 0)
    def _():
        counts_ref[...] += counts_new
        commit_ref[...] += loss_part

    @pl.when(i == _NB - 1)
    def _():
        loss_sum = commit_ref[0, 0]
        mse = loss_sum / jnp.float32(_B * _NC * _D)
        cb_ref[...] = jnp.full((1, 1), mse, dtype=jnp.float32)
        commit_ref[...] = jnp.full((1, 1), _COMMITMENT_COST * mse,
                                   dtype=jnp.float32)
        p = counts_ref[...] / jnp.float32(_B)
        ent = jnp.sum(p * jnp.log(p + 1e-10))
        perp_ref[...] = jnp.exp(-ent)[None, None]


def kernel(x, embedding):
    x_flat = x.reshape(-1, _NC, _D)
    x_sq = jnp.sum(x_flat ** 2, axis=-1)
    e_sq = jnp.sum(embedding ** 2, axis=-1)

    q, idx_t, _counts, commit, cb, perp = pl.pallas_call(
        _vq_block,
        grid=(_NB,),
        in_specs=[
            pl.BlockSpec((_TB, _NC, _D), lambda i: (i, 0, 0)),
            pl.BlockSpec((_NC, _M, _D), lambda i: (0, 0, 0)),
            pl.BlockSpec((_TB, _NC), lambda i: (i, 0)),
            pl.BlockSpec((_NC, _M), lambda i: (0, 0)),
        ],
        out_specs=[
            pl.BlockSpec((_TB, _NC, _D), lambda i: (i, 0, 0)),
            pl.BlockSpec((_NC, _TB), lambda i: (0, i)),
            pl.BlockSpec((_NC, _M), lambda i: (0, 0)),
            pl.BlockSpec((1, 1), lambda i: (0, 0)),
            pl.BlockSpec((1, 1), lambda i: (0, 0)),
            pl.BlockSpec((1, 1), lambda i: (0, 0)),
        ],
        out_shape=[
            jax.ShapeDtypeStruct((_B, _NC, _D), jnp.float32),
            jax.ShapeDtypeStruct((_NC, _B), jnp.int32),
            jax.ShapeDtypeStruct((_NC, _M), jnp.float32),
            jax.ShapeDtypeStruct((1, 1), jnp.float32),
            jax.ShapeDtypeStruct((1, 1), jnp.float32),
            jax.ShapeDtypeStruct((1, 1), jnp.float32),
        ],
    )(x_flat, embedding, x_sq, e_sq)

    quantized_out = q.reshape(x.shape)
    indices = idx_t.T
    return (quantized_out, commit[0, 0], cb[0, 0], perp[0, 0], indices)


# speedup vs baseline: 2.6682x; 2.6682x over previous
import jax
import jax.numpy as jnp
from jax import lax
from jax.experimental import pallas as pl
from jax.experimental.pallas import tpu as pltpu
from jax.experimental.pallas import tpu_sc as plsc

_NC = 4
_M = 4096
_D = 64
_B = 4608
_TB = 512
_NB = _B // _TB
_BN = _B * _NC
_NBINS = _NC * _M
_COMMITMENT_COST = 0.25

_SC_CORES = 2
_SC_SUBCORES = 16
_NW = _SC_CORES * _SC_SUBCORES
_RPW = _BN // _NW
_CH = 96
_NCH = _RPW // _CH


def _argmin_block(xm2_ref, e_ref, xsq_ref, esq_ref, idx_ref, loss_ref):
    i = pl.program_id(0)
    xb = xm2_ref[...]
    loss_part = jnp.zeros((1, 1), dtype=jnp.float32)
    rows = []
    for c in range(_NC):
        xc = xb[:, c, :]
        ec = e_ref[c]
        cross2 = lax.dot_general(
            xc, ec, (((1,), (1,)), ((), ())),
            preferred_element_type=jnp.float32)
        dist = (xsq_ref[:, c:c + 1] + cross2) + esq_ref[c][None, :]
        minv = jnp.min(dist, axis=-1)
        iota = lax.broadcasted_iota(jnp.int32, (_TB, _M), 1)
        idx_c = jnp.min(jnp.where(dist == minv[:, None], iota, _M), axis=-1)
        loss_part += jnp.sum(minv)[None, None]
        rows.append(idx_c)
    idx_ref[...] = jnp.stack(rows, axis=0)

    @pl.when(i == 0)
    def _():
        loss_ref[...] = loss_part

    @pl.when(i > 0)
    def _():
        loss_ref[...] += loss_part


def _sc_gather_hist(table_hbm, gidx_hbm, out_hbm, counts_hbm,
                    idx_v, rows_v, counts_v, sem):
    cid = lax.axis_index("c")
    sid = lax.axis_index("s")
    wid = sid * _SC_CORES + cid
    base = wid * _RPW
    pltpu.sync_copy(gidx_hbm.at[pl.ds(base, _RPW)], idx_v)
    copies = []
    for j in range(_NCH):
        copies.append(pltpu.async_copy(
            table_hbm.at[idx_v.at[pl.ds(j * _CH, _CH)]],
            rows_v.at[pl.ds(j * _CH, _CH)], sem))

    def _zero(k, carry):
        counts_v[pl.ds(k * 16, 16)] = jnp.zeros((16,), dtype=jnp.float32)
        return carry

    lax.fori_loop(0, _NBINS // 16, _zero, 0)
    ones = jnp.ones((16,), dtype=jnp.float32)

    def _hist(k, carry):
        idx16 = idx_v[pl.ds(k * 16, 16)]
        plsc.addupdate_scatter(counts_v, [idx16], ones)
        return carry

    lax.fori_loop(0, _RPW // 16, _hist, 0)

    for cpy in copies:
        cpy.wait()
    pltpu.sync_copy(rows_v, out_hbm.at[pl.ds(base, _RPW)])
    pltpu.sync_copy(counts_v, counts_hbm.at[wid])


_sc_gather_call = pl.kernel(
    _sc_gather_hist,
    out_type=[
        jax.ShapeDtypeStruct((_BN, _D), jnp.float32),
        jax.ShapeDtypeStruct((_NW, _NBINS), jnp.float32),
    ],
    mesh=plsc.VectorSubcoreMesh(
        core_axis_name="c", subcore_axis_name="s",
        num_cores=_SC_CORES, num_subcores=_SC_SUBCORES),
    scratch_types=[
        pltpu.VMEM((_RPW,), jnp.int32),
        pltpu.VMEM((_RPW, _D), jnp.float32),
        pltpu.VMEM((_NBINS,), jnp.float32),
        pltpu.SemaphoreType.DMA,
    ],
    compiler_params=pltpu.CompilerParams(needs_layout_passes=False,
                                         use_tc_tiling_on_sc=False),
)


def _finalize(counts_ref, loss_ref, commit_ref, cb_ref, perp_ref):
    counts = jnp.sum(counts_ref[...], axis=0)
    p = counts / jnp.float32(_B)
    ent = jnp.sum(p * jnp.log(p + 1e-10))
    perp_ref[...] = jnp.exp(-ent)[None, None]
    mse = loss_ref[0, 0] / jnp.float32(_BN * _D)
    cb_ref[...] = jnp.full((1, 1), mse, dtype=jnp.float32)
    commit_ref[...] = jnp.full((1, 1), _COMMITMENT_COST * mse,
                               dtype=jnp.float32)


def kernel(x, embedding):
    x_flat = x.reshape(_B, _NC, _D)
    xm2 = x_flat * (-2.0)
    x_sq = jnp.sum(x_flat ** 2, axis=-1)
    e_sq = jnp.sum(embedding ** 2, axis=-1)

    idx_t, loss = pl.pallas_call(
        _argmin_block,
        grid=(_NB,),
        in_specs=[
            pl.BlockSpec((_TB, _NC, _D), lambda i: (i, 0, 0)),
            pl.BlockSpec((_NC, _M, _D), lambda i: (0, 0, 0)),
            pl.BlockSpec((_TB, _NC), lambda i: (i, 0)),
            pl.BlockSpec((_NC, _M), lambda i: (0, 0)),
        ],
        out_specs=[
            pl.BlockSpec((_NC, _TB), lambda i: (0, i)),
            pl.BlockSpec((1, 1), lambda i: (0, 0)),
        ],
        out_shape=[
            jax.ShapeDtypeStruct((_NC, _B), jnp.int32),
            jax.ShapeDtypeStruct((1, 1), jnp.float32),
        ],
    )(xm2, embedding, x_sq, e_sq)

    indices = idx_t.T
    gidx = (indices
            + (jnp.arange(_NC, dtype=jnp.int32) * _M)[None, :]).reshape(_BN)
    table = embedding.reshape(_NBINS, _D)
    qrows, counts_part = _sc_gather_call(table, gidx)

    commit, cb, perp = pl.pallas_call(
        _finalize,
        out_shape=[
            jax.ShapeDtypeStruct((1, 1), jnp.float32),
            jax.ShapeDtypeStruct((1, 1), jnp.float32),
            jax.ShapeDtypeStruct((1, 1), jnp.float32),
        ],
    )(counts_part, loss)

    quantized_out = qrows.reshape(x.shape)
    return (quantized_out, commit[0, 0], cb[0, 0], perp[0, 0], indices)


# speedup vs baseline: 2.8062x; 1.0517x over previous
import jax
import jax.numpy as jnp
from jax import lax
from jax.experimental import pallas as pl
from jax.experimental.pallas import tpu as pltpu
from jax.experimental.pallas import tpu_sc as plsc

_NC = 4
_M = 4096
_D = 64
_B = 4608
_TB = 512
_NB = _B // _TB
_BN = _B * _NC
_NBINS = _NC * _M
_COMMITMENT_COST = 0.25

_SC_CORES = 2
_SC_SUBCORES = 16
_NW = _SC_CORES * _SC_SUBCORES
_RPW = _BN // _NW
_CH = 96
_NCH = _RPW // _CH


_MT = 1024
_NMT = _M // _MT


def _argmin_block(x_ref, e_ref, esq_ref, idx_ref, loss_ref):
    i = pl.program_id(0)
    xb = x_ref[...]
    loss_part = jnp.zeros((1, 1), dtype=jnp.float32)
    rows = []
    iota = lax.broadcasted_iota(jnp.int32, (_TB, _MT), 1)
    for c in range(_NC):
        xc = xb[:, c, :]
        xm2c = xc * (-2.0)
        xsq = jnp.sum(xc * xc, axis=-1)[:, None]
        minv = None
        idx_c = None
        for t in range(_NMT):
            et = e_ref[c, pl.ds(t * _MT, _MT), :]
            cross2 = lax.dot_general(
                xm2c, et, (((1,), (1,)), ((), ())),
                preferred_element_type=jnp.float32)
            dist = (xsq + cross2) + esq_ref[c, pl.ds(t * _MT, _MT)][None, :]
            tmin = jnp.min(dist, axis=-1)
            tidx = jnp.min(
                jnp.where(dist == tmin[:, None], iota + t * _MT, _M),
                axis=-1)
            if minv is None:
                minv, idx_c = tmin, tidx
            else:
                take = tmin < minv
                idx_c = jnp.where(take, tidx, idx_c)
                minv = jnp.minimum(tmin, minv)
        loss_part += jnp.sum(minv)[None, None]
        rows.append(idx_c)
    idx_ref[...] = jnp.stack(rows, axis=0)

    @pl.when(i == 0)
    def _():
        loss_ref[...] = loss_part

    @pl.when(i > 0)
    def _():
        loss_ref[...] += loss_part


def _sc_gather_hist(table_hbm, gidx_hbm, out_hbm, counts_hbm,
                    idx_v, rows_v, counts_v, sem):
    cid = lax.axis_index("c")
    sid = lax.axis_index("s")
    wid = sid * _SC_CORES + cid
    base = wid * _RPW
    pltpu.sync_copy(gidx_hbm.at[pl.ds(base, _RPW)], idx_v)
    copies = []
    for j in range(_NCH):
        copies.append(pltpu.async_copy(
            table_hbm.at[idx_v.at[pl.ds(j * _CH, _CH)]],
            rows_v.at[pl.ds(j * _CH, _CH)], sem))

    def _zero(k, carry):
        counts_v[pl.ds(k * 16, 16)] = jnp.zeros((16,), dtype=jnp.float32)
        return carry

    lax.fori_loop(0, _NBINS // 16, _zero, 0)
    ones = jnp.ones((16,), dtype=jnp.float32)

    def _hist(k, carry):
        idx16 = idx_v[pl.ds(k * 16, 16)]
        plsc.addupdate_scatter(counts_v, [idx16], ones)
        return carry

    lax.fori_loop(0, _RPW // 16, _hist, 0)

    for cpy in copies:
        cpy.wait()
    pltpu.sync_copy(rows_v, out_hbm.at[pl.ds(base, _RPW)])
    pltpu.sync_copy(counts_v, counts_hbm.at[wid])


_sc_gather_call = pl.kernel(
    _sc_gather_hist,
    out_type=[
        jax.ShapeDtypeStruct((_BN, _D), jnp.float32),
        jax.ShapeDtypeStruct((_NW, _NBINS), jnp.float32),
    ],
    mesh=plsc.VectorSubcoreMesh(
        core_axis_name="c", subcore_axis_name="s",
        num_cores=_SC_CORES, num_subcores=_SC_SUBCORES),
    scratch_types=[
        pltpu.VMEM((_RPW,), jnp.int32),
        pltpu.VMEM((_RPW, _D), jnp.float32),
        pltpu.VMEM((_NBINS,), jnp.float32),
        pltpu.SemaphoreType.DMA,
    ],
    compiler_params=pltpu.CompilerParams(needs_layout_passes=False,
                                         use_tc_tiling_on_sc=False),
)


def _finalize(counts_ref, loss_ref, commit_ref, cb_ref, perp_ref):
    counts = jnp.sum(counts_ref[...], axis=0)
    p = counts / jnp.float32(_B)
    ent = jnp.sum(p * jnp.log(p + 1e-10))
    perp_ref[...] = jnp.exp(-ent)[None, None]
    mse = loss_ref[0, 0] / jnp.float32(_BN * _D)
    cb_ref[...] = jnp.full((1, 1), mse, dtype=jnp.float32)
    commit_ref[...] = jnp.full((1, 1), _COMMITMENT_COST * mse,
                               dtype=jnp.float32)


def kernel(x, embedding):
    x_flat = x.reshape(_B, _NC, _D)
    e_sq = jnp.sum(embedding ** 2, axis=-1)

    idx_t, loss = pl.pallas_call(
        _argmin_block,
        grid=(_NB,),
        in_specs=[
            pl.BlockSpec((_TB, _NC, _D), lambda i: (i, 0, 0)),
            pl.BlockSpec((_NC, _M, _D), lambda i: (0, 0, 0)),
            pl.BlockSpec((_NC, _M), lambda i: (0, 0)),
        ],
        out_specs=[
            pl.BlockSpec((_NC, _TB), lambda i: (0, i)),
            pl.BlockSpec((1, 1), lambda i: (0, 0)),
        ],
        out_shape=[
            jax.ShapeDtypeStruct((_NC, _B), jnp.int32),
            jax.ShapeDtypeStruct((1, 1), jnp.float32),
        ],
    )(x_flat, embedding, e_sq)

    indices = idx_t.T
    gidx = (indices
            + (jnp.arange(_NC, dtype=jnp.int32) * _M)[None, :]).reshape(_BN)
    table = embedding.reshape(_NBINS, _D)
    qrows, counts_part = _sc_gather_call(table, gidx)

    commit, cb, perp = pl.pallas_call(
        _finalize,
        out_shape=[
            jax.ShapeDtypeStruct((1, 1), jnp.float32),
            jax.ShapeDtypeStruct((1, 1), jnp.float32),
            jax.ShapeDtypeStruct((1, 1), jnp.float32),
        ],
    )(counts_part, loss)

    quantized_out = qrows.reshape(x.shape)
    return (quantized_out, commit[0, 0], cb[0, 0], perp[0, 0], indices)


# speedup vs baseline: 3.4529x; 1.2305x over previous
import jax
import jax.numpy as jnp
from jax import lax
from jax.experimental import pallas as pl
from jax.experimental.pallas import tpu as pltpu
from jax.experimental.pallas import tpu_sc as plsc

_NC = 4
_M = 4096
_D = 64
_B = 4608
_TB = 512
_NB = _B // _TB
_BN = _B * _NC
_NBINS = _NC * _M
_COMMITMENT_COST = 0.25

_SC_CORES = 2
_SC_SUBCORES = 16
_NW = _SC_CORES * _SC_SUBCORES
_RPW = _BN // _NW
_CH = 96
_NCH = _RPW // _CH


_MT = 1024
_NMT = _M // _MT
_LW = 128
_NCH_M = _MT // _LW


def _argmin_block(x_ref, e_ref, esq_ref, idx_ref, loss_ref):
    i = pl.program_id(0)
    xb = x_ref[...]
    loss_part = jnp.zeros((1, 1), dtype=jnp.float32)
    rows = []
    lane = lax.broadcasted_iota(jnp.int32, (_TB, _LW), 1)
    for c in range(_NC):
        xc = xb[:, c, :]
        xm2c = xc * (-2.0)
        xsq = jnp.sum(xc * xc, axis=-1)[:, None]
        run_min = None
        run_chunk = None
        for t in range(_NMT):
            et = e_ref[c, pl.ds(t * _MT, _MT), :]
            cross2 = lax.dot_general(
                xm2c, et, (((1,), (1,)), ((), ())),
                preferred_element_type=jnp.float32)
            for s in range(_NCH_M):
                d = ((xsq + cross2[:, s * _LW:(s + 1) * _LW])
                     + esq_ref[c, pl.ds(t * _MT + s * _LW, _LW)][None, :])
                cid = t * _NCH_M + s
                if run_min is None:
                    run_min = d
                    run_chunk = jnp.zeros((_TB, _LW), dtype=jnp.int32)
                else:
                    take = d < run_min
                    run_chunk = jnp.where(take, cid, run_chunk)
                    run_min = jnp.minimum(d, run_min)
        combined = run_chunk * _LW + lane
        gmin = jnp.min(run_min, axis=-1)
        idx_c = jnp.min(
            jnp.where(run_min == gmin[:, None], combined, _M), axis=-1)
        loss_part += jnp.sum(gmin)[None, None]
        rows.append(idx_c)
    idx_ref[...] = jnp.stack(rows, axis=0)

    @pl.when(i == 0)
    def _():
        loss_ref[...] = loss_part

    @pl.when(i > 0)
    def _():
        loss_ref[...] += loss_part


def _sc_gather_hist(table_hbm, gidx_hbm, out_hbm, counts_hbm,
                    idx_v, rows_v, counts_v, sem):
    cid = lax.axis_index("c")
    sid = lax.axis_index("s")
    wid = sid * _SC_CORES + cid
    base = wid * _RPW
    pltpu.sync_copy(gidx_hbm.at[pl.ds(base, _RPW)], idx_v)
    copies = []
    for j in range(_NCH):
        copies.append(pltpu.async_copy(
            table_hbm.at[idx_v.at[pl.ds(j * _CH, _CH)]],
            rows_v.at[pl.ds(j * _CH, _CH)], sem))

    def _zero(k, carry):
        counts_v[pl.ds(k * 16, 16)] = jnp.zeros((16,), dtype=jnp.float32)
        return carry

    lax.fori_loop(0, _NBINS // 16, _zero, 0)
    ones = jnp.ones((16,), dtype=jnp.float32)

    def _hist(k, carry):
        idx16 = idx_v[pl.ds(k * 16, 16)]
        plsc.addupdate_scatter(counts_v, [idx16], ones)
        return carry

    lax.fori_loop(0, _RPW // 16, _hist, 0)

    for cpy in copies:
        cpy.wait()
    pltpu.sync_copy(rows_v, out_hbm.at[pl.ds(base, _RPW)])
    pltpu.sync_copy(counts_v, counts_hbm.at[wid])


_sc_gather_call = pl.kernel(
    _sc_gather_hist,
    out_type=[
        jax.ShapeDtypeStruct((_BN, _D), jnp.float32),
        jax.ShapeDtypeStruct((_NW, _NBINS), jnp.float32),
    ],
    mesh=plsc.VectorSubcoreMesh(
        core_axis_name="c", subcore_axis_name="s",
        num_cores=_SC_CORES, num_subcores=_SC_SUBCORES),
    scratch_types=[
        pltpu.VMEM((_RPW,), jnp.int32),
        pltpu.VMEM((_RPW, _D), jnp.float32),
        pltpu.VMEM((_NBINS,), jnp.float32),
        pltpu.SemaphoreType.DMA,
    ],
    compiler_params=pltpu.CompilerParams(needs_layout_passes=False,
                                         use_tc_tiling_on_sc=False),
)


def _finalize(counts_ref, loss_ref, commit_ref, cb_ref, perp_ref):
    counts = jnp.sum(counts_ref[...], axis=0)
    p = counts / jnp.float32(_B)
    ent = jnp.sum(p * jnp.log(p + 1e-10))
    perp_ref[...] = jnp.exp(-ent)[None, None]
    mse = loss_ref[0, 0] / jnp.float32(_BN * _D)
    cb_ref[...] = jnp.full((1, 1), mse, dtype=jnp.float32)
    commit_ref[...] = jnp.full((1, 1), _COMMITMENT_COST * mse,
                               dtype=jnp.float32)


def kernel(x, embedding):
    x_flat = x.reshape(_B, _NC, _D)
    e_sq = jnp.sum(embedding ** 2, axis=-1)

    idx_t, loss = pl.pallas_call(
        _argmin_block,
        grid=(_NB,),
        in_specs=[
            pl.BlockSpec((_TB, _NC, _D), lambda i: (i, 0, 0)),
            pl.BlockSpec((_NC, _M, _D), lambda i: (0, 0, 0)),
            pl.BlockSpec((_NC, _M), lambda i: (0, 0)),
        ],
        out_specs=[
            pl.BlockSpec((_NC, _TB), lambda i: (0, i)),
            pl.BlockSpec((1, 1), lambda i: (0, 0)),
        ],
        out_shape=[
            jax.ShapeDtypeStruct((_NC, _B), jnp.int32),
            jax.ShapeDtypeStruct((1, 1), jnp.float32),
        ],
    )(x_flat, embedding, e_sq)

    indices = idx_t.T
    gidx = (indices
            + (jnp.arange(_NC, dtype=jnp.int32) * _M)[None, :]).reshape(_BN)
    table = embedding.reshape(_NBINS, _D)
    qrows, counts_part = _sc_gather_call(table, gidx)

    commit, cb, perp = pl.pallas_call(
        _finalize,
        out_shape=[
            jax.ShapeDtypeStruct((1, 1), jnp.float32),
            jax.ShapeDtypeStruct((1, 1), jnp.float32),
            jax.ShapeDtypeStruct((1, 1), jnp.float32),
        ],
    )(counts_part, loss)

    quantized_out = qrows.reshape(x.shape)
    return (quantized_out, commit[0, 0], cb[0, 0], perp[0, 0], indices)
